# baseline (speedup 1.0000x reference)
import functools

import jax
import jax.numpy as jnp
from jax import lax
from jax.experimental import pallas as pl
from jax.experimental.pallas import tpu as pltpu
from jax.experimental.pallas import tpu_sc as plsc

N_NODES = 10000
N_EDGES = 320000
IN_DIM = 128
HID_DIM = 64
OUT_DIM = 3

W1P = 80
W2P = 16

NC = 2
NS = 16
NW = NC * NS
CHUNK = 128
NCH = N_EDGES // CHUNK
NCHW = -(-NCH // NW)
NCH_PAD = NW * NCHW
N_TAB = N_NODES + 8
STRIPE = 624
TAIL = N_NODES - NS * STRIPE


def _lin1_body(x_ref, wl_ref, wr_ref, b_ref, y1p_ref, xr1_ref):
    x = x_ref[...]
    y = lax.dot_general(x, wl_ref[...], (((1,), (1,)), ((), ())),
                        preferred_element_type=jnp.float32)
    n = x.shape[0]
    top = jnp.concatenate(
        [y, jnp.ones((n, 1), jnp.float32),
         jnp.zeros((n, W1P - HID_DIM - 1), jnp.float32)], axis=1)
    y1p_ref[...] = jnp.concatenate(
        [top, jnp.zeros((N_TAB - n, W1P), jnp.float32)], axis=0)
    xr1_ref[...] = lax.dot_general(x, wr_ref[...], (((1,), (1,)), ((), ())),
                                   preferred_element_type=jnp.float32) + b_ref[...]


def _lin1(x, wl, wr, b):
    return pl.pallas_call(
        _lin1_body,
        out_shape=(
            jax.ShapeDtypeStruct((N_TAB, W1P), jnp.float32),
            jax.ShapeDtypeStruct((N_NODES, HID_DIM), jnp.float32),
        ),
    )(x, wl, wr, b)


def _lin2_body(aggp_ref, xr1_ref, wl_ref, wr_ref, b_ref,
               y2p_ref, r2_ref, dinv_ref):
    agg = aggp_ref[0] + aggp_ref[1]
    deg = agg[:, HID_DIM:HID_DIM + 1]
    dinv = 1.0 / jnp.maximum(deg, 1.0)
    h = jnp.maximum(agg[:, :HID_DIM] * dinv + xr1_ref[...], 0.0)
    y2 = lax.dot_general(h, wl_ref[...], (((1,), (1,)), ((), ())),
                         preferred_element_type=jnp.float32)
    n = h.shape[0]
    top = jnp.concatenate(
        [y2, jnp.zeros((n, W2P - OUT_DIM), jnp.float32)], axis=1)
    y2p_ref[...] = jnp.concatenate(
        [top, jnp.zeros((N_TAB - n, W2P), jnp.float32)], axis=0)
    r2_ref[...] = lax.dot_general(h, wr_ref[...], (((1,), (1,)), ((), ())),
                                  preferred_element_type=jnp.float32) + b_ref[...]
    dinv_ref[...] = dinv


def _lin2(aggp, xr1, wl, wr, b):
    return pl.pallas_call(
        _lin2_body,
        out_shape=(
            jax.ShapeDtypeStruct((N_TAB, W2P), jnp.float32),
            jax.ShapeDtypeStruct((N_NODES, OUT_DIM), jnp.float32),
            jax.ShapeDtypeStruct((N_NODES, 1), jnp.float32),
        ),
    )(aggp, xr1, wl, wr, b)


def _final_body(aggp_ref, r2_ref, dinv_ref, o_ref):
    agg = aggp_ref[0] + aggp_ref[1]
    z = agg[:, :OUT_DIM] * dinv_ref[...] + r2_ref[...]
    m = jnp.max(z, axis=1, keepdims=True)
    lse = jnp.log(jnp.sum(jnp.exp(z - m), axis=1, keepdims=True)) + m
    o_ref[...] = z - lse


def _final(aggp, r2, dinv):
    return pl.pallas_call(
        _final_body,
        out_shape=jax.ShapeDtypeStruct((N_NODES, OUT_DIM), jnp.float32),
    )(aggp, r2, dinv)


def _make_scatter(width, nch_c0, nch_c1):
    assert nch_c0 % 4 == 3 and nch_c1 % 4 == 3
    assert NS * (nch_c0 + nch_c1) == NCH_PAD
    nmax = max(nch_c0, nch_c1)
    mesh = plsc.VectorSubcoreMesh(core_axis_name="c", subcore_axis_name="s")

    pieces = [(k * CHUNK, CHUNK) for k in range(STRIPE // CHUNK)]
    if STRIPE % CHUNK:
        pieces.append((STRIPE - STRIPE % CHUNK, STRIPE % CHUNK))

    @functools.partial(
        pl.kernel,
        mesh=mesh,
        compiler_params=pltpu.CompilerParams(use_tc_tiling_on_sc=False),
        out_type=jax.ShapeDtypeStruct((NC, N_NODES, width), jnp.float32),
        scratch_types=[
            pltpu.VMEM((nmax, CHUNK), jnp.int32),
            pltpu.VMEM((nmax, CHUNK), jnp.int32),
            pltpu.VMEM((CHUNK, width), jnp.float32),
            pltpu.VMEM((CHUNK, width), jnp.float32),
            pltpu.VMEM((CHUNK, width), jnp.float32),
            pltpu.VMEM((CHUNK, width), jnp.float32),
            pltpu.VMEM_SHARED((N_NODES, width), jnp.float32),
            pltpu.SemaphoreType.DMA,
            pltpu.SemaphoreType.DMA,
            pltpu.SemaphoreType.DMA,
            pltpu.SemaphoreType.DMA,
            pltpu.SemaphoreType.DMA,
            pltpu.SemaphoreType.DMA,
            pltpu.SemaphoreType.DMA,
            pltpu.SemaphoreType.DMA,
            pltpu.SemaphoreType.DMA,
            pltpu.SemaphoreType.DMA,
        ],
    )
    def scat(edges_hbm, tab_hbm, out_hbm,
             src_v, dst_v, rows0_v, rows1_v, rows2_v, rows3_v, acc_sh,
             isem, zsem, g0, g1, g2, g3, s0, s1, s2, s3):
        c = lax.axis_index("c")
        s = lax.axis_index("s")
        r0 = s * STRIPE
        gsems = (g0, g1, g2, g3)
        ssems = (s0, s1, s2, s3)
        rows = (rows0_v, rows1_v, rows2_v, rows3_v)

        def preload(nchw, ch0):
            cp_s = pltpu.async_copy(edges_hbm.at[0, pl.ds(ch0, nchw)],
                                    src_v.at[pl.ds(0, nchw)], isem)
            cp_d = pltpu.async_copy(edges_hbm.at[1, pl.ds(ch0, nchw)],
                                    dst_v.at[pl.ds(0, nchw)], isem)
            cp_s.wait()
            cp_d.wait()

        with jax.named_scope("sc_preload"):
            @pl.when(c == 0)
            def _():
                preload(nch_c0, s * nch_c0)

            @pl.when(c == 1)
            def _():
                preload(nch_c1, NS * nch_c0 + s * nch_c1)

        def zero_body(r, _):
            for k in range(width // 16):
                rows0_v[r, pl.ds(k * 16, 16)] = jnp.zeros((16,), jnp.float32)
            return 0

        with jax.named_scope("sc_zerofill"):
            lax.fori_loop(0, CHUNK, zero_body, 0)
        for (o, n) in pieces:
            pltpu.async_copy(rows0_v.at[pl.ds(0, n)],
                             acc_sh.at[pl.ds(r0 + o, n)], zsem)

        @pl.when(s == NS - 1)
        def _():
            pltpu.async_copy(rows0_v.at[pl.ds(0, TAIL)],
                             acc_sh.at[pl.ds(NS * STRIPE, TAIL)], zsem)

        for (o, n) in pieces:
            pltpu.make_async_copy(rows0_v.at[pl.ds(0, n)],
                                  acc_sh.at[pl.ds(r0 + o, n)], zsem).wait()

        @pl.when(s == NS - 1)
        def _():
            pltpu.make_async_copy(rows0_v.at[pl.ds(0, TAIL)],
                                  acc_sh.at[pl.ds(NS * STRIPE, TAIL)],
                                  zsem).wait()

        with jax.named_scope("sc_barrier1"):
            plsc.subcore_barrier()

        def g_fire(j, b):
            pltpu.async_copy(tab_hbm.at[src_v.at[j]], rows[b], gsems[b])

        def g_wait(j, b):
            pltpu.make_async_copy(tab_hbm.at[src_v.at[j]], rows[b],
                                  gsems[b]).wait()

        def s_fire(j, b):
            pltpu.async_copy(rows[b], acc_sh.at[dst_v.at[j]], ssems[b],
                             add=True)

        def s_wait(j, b):
            pltpu.make_async_copy(rows[b], acc_sh.at[dst_v.at[j]],
                                  ssems[b]).wait()

        def ring(nchw):
            g_fire(0, 0)
            g_fire(1, 1)
            g_fire(2, 2)
            for j in range(4):
                g_wait(j, j % 4)
                s_fire(j, j % 4)
                if j >= 1:
                    s_wait(j - 1, (j - 1) % 4)
                g_fire(j + 3, (j + 3) % 4)

            def quad_body(g, _):
                for b in range(4):
                    j = g * 4 + b
                    g_wait(j, b)
                    s_fire(j, b)
                    s_wait(j - 1, (b - 1) % 4)
                    g_fire(j + 3, (b + 3) % 4)
                return 0

            lax.fori_loop(1, (nchw - 3) // 4, quad_body, 0)
            for j in range(nchw - 3, nchw):
                g_wait(j, j % 4)
                s_fire(j, j % 4)
                s_wait(j - 1, (j - 1) % 4)
            s_wait(nchw - 1, (nchw - 1) % 4)

        with jax.named_scope("sc_ring"):
            @pl.when(c == 0)
            def _():
                ring(nch_c0)

            @pl.when(c == 1)
            def _():
                ring(nch_c1)

        with jax.named_scope("sc_barrier2"):
            plsc.subcore_barrier()

        def in_hop(i):
            o, n = pieces[i]
            return (acc_sh.at[pl.ds(r0 + o, n)], rows[i % 2].at[pl.ds(0, n)])

        def out_hop(i):
            o, n = pieces[i]
            return (rows[i % 2].at[pl.ds(0, n)],
                    out_hbm.at[c, pl.ds(r0 + o, n)])

        scope_out = jax.named_scope("sc_copyout"); scope_out.__enter__()
        pltpu.async_copy(*in_hop(0), zsem)
        for i in range(len(pieces)):
            pltpu.make_async_copy(*in_hop(i), zsem).wait()
            pltpu.async_copy(*out_hop(i), ssems[i % 2])
            if i >= 1:
                pltpu.make_async_copy(*out_hop(i - 1), ssems[(i - 1) % 2]).wait()
            if i + 1 < len(pieces):
                pltpu.async_copy(*in_hop(i + 1), zsem)
        last = len(pieces) - 1
        pltpu.make_async_copy(*out_hop(last), ssems[last % 2]).wait()

        @pl.when(s == NS - 1)
        def _():
            pltpu.sync_copy(acc_sh.at[pl.ds(NS * STRIPE, TAIL)],
                            rows2_v.at[pl.ds(0, TAIL)])
            pltpu.sync_copy(rows2_v.at[pl.ds(0, TAIL)],
                            out_hbm.at[c, pl.ds(NS * STRIPE, TAIL)])
        scope_out.__exit__(None, None, None)

    return scat


_scatter1 = _make_scatter(W1P, 79, 79)
_scatter2 = _make_scatter(W2P, 79, 79)


def kernel(x, edge_index, W1_l, b1, W1_r, W2_l, b2, W2_r):
    padn = NCH_PAD * CHUNK - N_EDGES
    pad = jnp.concatenate(
        [jnp.full((1, padn), N_NODES, jnp.int32),
         jnp.zeros((1, padn), jnp.int32)], axis=0)
    edges3 = jnp.concatenate([edge_index, pad], axis=1).reshape(
        2, NCH_PAD, CHUNK)
    y1p, xr1 = _lin1(x, W1_l, W1_r, b1.reshape(1, HID_DIM))
    agg1p = _scatter1(edges3, y1p)
    y2p, r2, dinv = _lin2(agg1p, xr1, W2_l, W2_r, b2.reshape(1, OUT_DIM))
    agg2p = _scatter2(edges3, y2p)
    return _final(agg2p, r2, dinv)


# speedup vs baseline: 1.0031x; 1.0031x over previous
import functools

import jax
import jax.numpy as jnp
from jax import lax
from jax.experimental import pallas as pl
from jax.experimental.pallas import tpu as pltpu
from jax.experimental.pallas import tpu_sc as plsc

N_NODES = 10000
N_EDGES = 320000
IN_DIM = 128
HID_DIM = 64
OUT_DIM = 3

W1P = 80
W2P = 16

NC = 2
NS = 16
NW = NC * NS
CHUNK = 128
NCH = N_EDGES // CHUNK
NCHW = -(-NCH // NW)
NCH_PAD = NW * NCHW
N_TAB = N_NODES + 8
STRIPE = 624
TAIL = N_NODES - NS * STRIPE


def _lin1_body(x_ref, wl_ref, wr_ref, b_ref, y1p_ref, xr1_ref):
    x = x_ref[...]
    y = lax.dot_general(x, wl_ref[...], (((1,), (1,)), ((), ())),
                        preferred_element_type=jnp.float32)
    n = x.shape[0]
    top = jnp.concatenate(
        [y, jnp.ones((n, 1), jnp.float32),
         jnp.zeros((n, W1P - HID_DIM - 1), jnp.float32)], axis=1)
    y1p_ref[...] = jnp.concatenate(
        [top, jnp.zeros((N_TAB - n, W1P), jnp.float32)], axis=0)
    xr1_ref[...] = lax.dot_general(x, wr_ref[...], (((1,), (1,)), ((), ())),
                                   preferred_element_type=jnp.float32) + b_ref[...]


def _lin1(x, wl, wr, b):
    return pl.pallas_call(
        _lin1_body,
        out_shape=(
            jax.ShapeDtypeStruct((N_TAB, W1P), jnp.float32),
            jax.ShapeDtypeStruct((N_NODES, HID_DIM), jnp.float32),
        ),
    )(x, wl, wr, b)


def _lin2_body(aggp_ref, xr1_ref, wl_ref, wr_ref, b_ref,
               y2p_ref, r2_ref, dinv_ref):
    agg = aggp_ref[0] + aggp_ref[1]
    deg = agg[:, HID_DIM:HID_DIM + 1]
    dinv = 1.0 / jnp.maximum(deg, 1.0)
    h = jnp.maximum(agg[:, :HID_DIM] * dinv + xr1_ref[...], 0.0)
    y2 = lax.dot_general(h, wl_ref[...], (((1,), (1,)), ((), ())),
                         preferred_element_type=jnp.float32)
    n = h.shape[0]
    top = jnp.concatenate(
        [y2, jnp.zeros((n, W2P - OUT_DIM), jnp.float32)], axis=1)
    y2p_ref[...] = jnp.concatenate(
        [top, jnp.zeros((N_TAB - n, W2P), jnp.float32)], axis=0)
    r2_ref[...] = lax.dot_general(h, wr_ref[...], (((1,), (1,)), ((), ())),
                                  preferred_element_type=jnp.float32) + b_ref[...]
    dinv_ref[...] = dinv


def _lin2(aggp, xr1, wl, wr, b):
    return pl.pallas_call(
        _lin2_body,
        out_shape=(
            jax.ShapeDtypeStruct((N_TAB, W2P), jnp.float32),
            jax.ShapeDtypeStruct((N_NODES, OUT_DIM), jnp.float32),
            jax.ShapeDtypeStruct((N_NODES, 1), jnp.float32),
        ),
    )(aggp, xr1, wl, wr, b)


def _final_body(aggp_ref, r2_ref, dinv_ref, o_ref):
    agg = aggp_ref[0] + aggp_ref[1]
    z = agg[:, :OUT_DIM] * dinv_ref[...] + r2_ref[...]
    m = jnp.max(z, axis=1, keepdims=True)
    lse = jnp.log(jnp.sum(jnp.exp(z - m), axis=1, keepdims=True)) + m
    o_ref[...] = z - lse


def _final(aggp, r2, dinv):
    return pl.pallas_call(
        _final_body,
        out_shape=jax.ShapeDtypeStruct((N_NODES, OUT_DIM), jnp.float32),
    )(aggp, r2, dinv)


def _make_scatter(width, nch_c0, nch_c1):
    assert nch_c0 % 4 == 3 and nch_c1 % 4 == 3
    assert NS * (nch_c0 + nch_c1) == NCH_PAD
    nmax = max(nch_c0, nch_c1)
    mesh = plsc.VectorSubcoreMesh(core_axis_name="c", subcore_axis_name="s")

    pieces = [(k * CHUNK, CHUNK) for k in range(STRIPE // CHUNK)]
    if STRIPE % CHUNK:
        pieces.append((STRIPE - STRIPE % CHUNK, STRIPE % CHUNK))

    @functools.partial(
        pl.kernel,
        mesh=mesh,
        compiler_params=pltpu.CompilerParams(use_tc_tiling_on_sc=False),
        out_type=jax.ShapeDtypeStruct((NC, N_NODES, width), jnp.float32),
        scratch_types=[
            pltpu.VMEM((nmax, CHUNK), jnp.int32),
            pltpu.VMEM((nmax, CHUNK), jnp.int32),
            pltpu.VMEM((CHUNK, width), jnp.float32),
            pltpu.VMEM((CHUNK, width), jnp.float32),
            pltpu.VMEM((CHUNK, width), jnp.float32),
            pltpu.VMEM((CHUNK, width), jnp.float32),
            pltpu.VMEM_SHARED((N_NODES, width), jnp.float32),
            pltpu.SemaphoreType.DMA,
            pltpu.SemaphoreType.DMA,
            pltpu.SemaphoreType.DMA,
            pltpu.SemaphoreType.DMA,
            pltpu.SemaphoreType.DMA,
            pltpu.SemaphoreType.DMA,
            pltpu.SemaphoreType.DMA,
            pltpu.SemaphoreType.DMA,
            pltpu.SemaphoreType.DMA,
            pltpu.SemaphoreType.DMA,
        ],
    )
    def scat(edges_hbm, tab_hbm, out_hbm,
             src_v, dst_v, rows0_v, rows1_v, rows2_v, rows3_v, acc_sh,
             isem, zsem, g0, g1, g2, g3, s0, s1, s2, s3):
        c = lax.axis_index("c")
        s = lax.axis_index("s")
        r0 = s * STRIPE
        gsems = (g0, g1, g2, g3)
        ssems = (s0, s1, s2, s3)
        rows = (rows0_v, rows1_v, rows2_v, rows3_v)

        def preload(nchw, ch0):
            cp_s = pltpu.async_copy(edges_hbm.at[0, pl.ds(ch0, nchw)],
                                    src_v.at[pl.ds(0, nchw)], isem)
            cp_d = pltpu.async_copy(edges_hbm.at[1, pl.ds(ch0, nchw)],
                                    dst_v.at[pl.ds(0, nchw)], isem)
            cp_s.wait()
            cp_d.wait()

        with jax.named_scope("sc_preload"):
            @pl.when(c == 0)
            def _():
                preload(nch_c0, s * nch_c0)

            @pl.when(c == 1)
            def _():
                preload(nch_c1, NS * nch_c0 + s * nch_c1)

        def zero_body(r, _):
            for k in range(width // 16):
                rows0_v[r, pl.ds(k * 16, 16)] = jnp.zeros((16,), jnp.float32)
            return 0

        with jax.named_scope("sc_zerofill"):
            lax.fori_loop(0, CHUNK, zero_body, 0)
        for (o, n) in pieces:
            pltpu.async_copy(rows0_v.at[pl.ds(0, n)],
                             acc_sh.at[pl.ds(r0 + o, n)], zsem)

        @pl.when(s == NS - 1)
        def _():
            pltpu.async_copy(rows0_v.at[pl.ds(0, TAIL)],
                             acc_sh.at[pl.ds(NS * STRIPE, TAIL)], zsem)

        for (o, n) in pieces:
            pltpu.make_async_copy(rows0_v.at[pl.ds(0, n)],
                                  acc_sh.at[pl.ds(r0 + o, n)], zsem).wait()

        @pl.when(s == NS - 1)
        def _():
            pltpu.make_async_copy(rows0_v.at[pl.ds(0, TAIL)],
                                  acc_sh.at[pl.ds(NS * STRIPE, TAIL)],
                                  zsem).wait()

        with jax.named_scope("sc_barrier1"):
            plsc.subcore_barrier()

        def g_fire(j, b):
            pltpu.async_copy(tab_hbm.at[src_v.at[j]], rows[b], gsems[b])

        def g_wait(j, b):
            pltpu.make_async_copy(tab_hbm.at[src_v.at[j]], rows[b],
                                  gsems[b]).wait()

        def s_fire(j, b):
            pltpu.async_copy(rows[b], acc_sh.at[dst_v.at[j]], ssems[b],
                             add=True)

        def s_wait(j, b):
            pltpu.make_async_copy(rows[b], acc_sh.at[dst_v.at[j]],
                                  ssems[b]).wait()

        def ring(nchw):
            g_fire(0, 0)
            g_fire(1, 1)
            g_fire(2, 2)
            for j in range(4):
                g_wait(j, j % 4)
                s_fire(j, j % 4)
                if j >= 1:
                    s_wait(j - 1, (j - 1) % 4)
                g_fire(j + 3, (j + 3) % 4)

            def quad_body(g, _):
                for b in range(4):
                    j = g * 4 + b
                    g_wait(j, b)
                    s_fire(j, b)
                    s_wait(j - 1, (b - 1) % 4)
                    g_fire(j + 3, (b + 3) % 4)
                return 0

            lax.fori_loop(1, (nchw - 3) // 4, quad_body, 0)
            for j in range(nchw - 3, nchw):
                g_wait(j, j % 4)
                s_fire(j, j % 4)
                s_wait(j - 1, (j - 1) % 4)
            s_wait(nchw - 1, (nchw - 1) % 4)

        with jax.named_scope("sc_ring"):
            @pl.when(c == 0)
            def _():
                ring(nch_c0)

            @pl.when(c == 1)
            def _():
                ring(nch_c1)

        with jax.named_scope("sc_barrier2"):
            plsc.subcore_barrier()

        def in_hop(i):
            o, n = pieces[i]
            return (acc_sh.at[pl.ds(r0 + o, n)], rows[i % 2].at[pl.ds(0, n)])

        def out_hop(i):
            o, n = pieces[i]
            return (rows[i % 2].at[pl.ds(0, n)],
                    out_hbm.at[c, pl.ds(r0 + o, n)])

        scope_out = jax.named_scope("sc_copyout"); scope_out.__enter__()
        pltpu.async_copy(*in_hop(0), zsem)
        for i in range(len(pieces)):
            pltpu.make_async_copy(*in_hop(i), zsem).wait()
            pltpu.async_copy(*out_hop(i), ssems[i % 2])
            if i >= 1:
                pltpu.make_async_copy(*out_hop(i - 1), ssems[(i - 1) % 2]).wait()
            if i + 1 < len(pieces):
                pltpu.async_copy(*in_hop(i + 1), zsem)
        last = len(pieces) - 1
        pltpu.make_async_copy(*out_hop(last), ssems[last % 2]).wait()

        @pl.when(s == NS - 1)
        def _():
            pltpu.sync_copy(acc_sh.at[pl.ds(NS * STRIPE, TAIL)],
                            rows2_v.at[pl.ds(0, TAIL)])
            pltpu.sync_copy(rows2_v.at[pl.ds(0, TAIL)],
                            out_hbm.at[c, pl.ds(NS * STRIPE, TAIL)])
        scope_out.__exit__(None, None, None)

    return scat


_scatter1 = _make_scatter(W1P, 79, 79)
_scatter2 = _make_scatter(W2P, 79, 79)


def kernel(x, edge_index, W1_l, b1, W1_r, W2_l, b2, W2_r):
    padn = NCH_PAD * CHUNK - N_EDGES
    pad = jnp.concatenate(
        [jnp.full((1, padn), N_NODES, jnp.int32),
         (jnp.arange(padn, dtype=jnp.int32) % N_NODES).reshape(1, padn)],
        axis=0)
    edges3 = jnp.concatenate([edge_index, pad], axis=1).reshape(
        2, NCH_PAD, CHUNK)
    y1p, xr1 = _lin1(x, W1_l, W1_r, b1.reshape(1, HID_DIM))
    agg1p = _scatter1(edges3, y1p)
    y2p, r2, dinv = _lin2(agg1p, xr1, W2_l, W2_r, b2.reshape(1, OUT_DIM))
    agg2p = _scatter2(edges3, y2p)
    return _final(agg2p, r2, dinv)


# speedup vs baseline: 1.6372x; 1.6322x over previous
import functools

import jax
import jax.numpy as jnp
from jax import lax
from jax.experimental import pallas as pl
from jax.experimental.pallas import tpu as pltpu
from jax.experimental.pallas import tpu_sc as plsc

N_NODES = 10000
N_EDGES = 320000
IN_DIM = 128
HID_DIM = 64
OUT_DIM = 3

W1P = 80
W2P = 16

NC = 2
NS = 16
NW = NC * NS
CHUNK = 128
NCH = N_EDGES // CHUNK
NCHW = -(-NCH // NW)
NCH_PAD = NW * NCHW
N_PAD_EDGES = NW * (-(-(N_EDGES // 128) // NW)) * 128 - N_EDGES
N_TAB = N_NODES + 8
STRIPE = 624
TAIL = N_NODES - NS * STRIPE


def _lin1_body(x_ref, wl_ref, wr_ref, b_ref, y1p_ref, xr1_ref):
    x = x_ref[...]
    y = lax.dot_general(x, wl_ref[...], (((1,), (1,)), ((), ())),
                        preferred_element_type=jnp.float32)
    n = x.shape[0]
    top = jnp.concatenate(
        [y, jnp.ones((n, 1), jnp.float32),
         jnp.zeros((n, W1P - HID_DIM - 1), jnp.float32)], axis=1)
    y1p_ref[...] = jnp.concatenate(
        [top, jnp.zeros((N_TAB - n, W1P), jnp.float32)], axis=0)
    xr1_ref[...] = lax.dot_general(x, wr_ref[...], (((1,), (1,)), ((), ())),
                                   preferred_element_type=jnp.float32) + b_ref[...]


def _lin1(x, wl, wr, b):
    return pl.pallas_call(
        _lin1_body,
        out_shape=(
            jax.ShapeDtypeStruct((N_TAB, W1P), jnp.float32),
            jax.ShapeDtypeStruct((N_NODES, HID_DIM), jnp.float32),
        ),
    )(x, wl, wr, b)


def _lin2_body(aggp_ref, xr1_ref, wl_ref, wr_ref, b_ref,
               y2p_ref, r2_ref, dinv_ref):
    agg = aggp_ref[0] + aggp_ref[1]
    deg = agg[:, HID_DIM:HID_DIM + 1]
    dinv = 1.0 / jnp.maximum(deg, 1.0)
    h = jnp.maximum(agg[:, :HID_DIM] * dinv + xr1_ref[...], 0.0)
    y2 = lax.dot_general(h, wl_ref[...], (((1,), (1,)), ((), ())),
                         preferred_element_type=jnp.float32)
    n = h.shape[0]
    top = jnp.concatenate(
        [y2, jnp.zeros((n, W2P - OUT_DIM), jnp.float32)], axis=1)
    y2p_ref[...] = jnp.concatenate(
        [top, jnp.zeros((N_TAB - n, W2P), jnp.float32)], axis=0)
    r2_ref[...] = lax.dot_general(h, wr_ref[...], (((1,), (1,)), ((), ())),
                                  preferred_element_type=jnp.float32) + b_ref[...]
    dinv_ref[...] = dinv


def _lin2(aggp, xr1, wl, wr, b):
    return pl.pallas_call(
        _lin2_body,
        out_shape=(
            jax.ShapeDtypeStruct((N_TAB, W2P), jnp.float32),
            jax.ShapeDtypeStruct((N_NODES, OUT_DIM), jnp.float32),
            jax.ShapeDtypeStruct((N_NODES, 1), jnp.float32),
        ),
    )(aggp, xr1, wl, wr, b)


def _final_body(aggp_ref, r2_ref, dinv_ref, o_ref):
    agg = aggp_ref[0] + aggp_ref[1]
    z = agg[:, :OUT_DIM] * dinv_ref[...] + r2_ref[...]
    m = jnp.max(z, axis=1, keepdims=True)
    lse = jnp.log(jnp.sum(jnp.exp(z - m), axis=1, keepdims=True)) + m
    o_ref[...] = z - lse


def _final(aggp, r2, dinv):
    return pl.pallas_call(
        _final_body,
        out_shape=jax.ShapeDtypeStruct((N_NODES, OUT_DIM), jnp.float32),
    )(aggp, r2, dinv)


def _make_scatter(width, nch_c0, nch_c1):
    assert nch_c0 % 4 == 3 and nch_c1 % 4 == 3
    assert NS * (nch_c0 + nch_c1) == NCH_PAD
    nmax = max(nch_c0, nch_c1)
    mesh = plsc.VectorSubcoreMesh(core_axis_name="c", subcore_axis_name="s")

    pieces = [(k * CHUNK, CHUNK) for k in range(STRIPE // CHUNK)]
    if STRIPE % CHUNK:
        pieces.append((STRIPE - STRIPE % CHUNK, STRIPE % CHUNK))

    @functools.partial(
        pl.kernel,
        mesh=mesh,
        compiler_params=pltpu.CompilerParams(use_tc_tiling_on_sc=False),
        out_type=jax.ShapeDtypeStruct((NC, N_NODES, width), jnp.float32),
        scratch_types=[
            pltpu.VMEM((nmax, CHUNK), jnp.int32),
            pltpu.VMEM((nmax, CHUNK), jnp.int32),
            pltpu.VMEM((CHUNK, width), jnp.float32),
            pltpu.VMEM((CHUNK, width), jnp.float32),
            pltpu.VMEM((CHUNK, width), jnp.float32),
            pltpu.VMEM((CHUNK, width), jnp.float32),
            pltpu.VMEM_SHARED((N_NODES + N_PAD_EDGES, width), jnp.float32),
            pltpu.SemaphoreType.DMA,
            pltpu.SemaphoreType.DMA,
            pltpu.SemaphoreType.DMA,
            pltpu.SemaphoreType.DMA,
            pltpu.SemaphoreType.DMA,
            pltpu.SemaphoreType.DMA,
            pltpu.SemaphoreType.DMA,
            pltpu.SemaphoreType.DMA,
            pltpu.SemaphoreType.DMA,
            pltpu.SemaphoreType.DMA,
        ],
    )
    def scat(edges_hbm, tab_hbm, out_hbm,
             src_v, dst_v, rows0_v, rows1_v, rows2_v, rows3_v, acc_sh,
             isem, zsem, g0, g1, g2, g3, s0, s1, s2, s3):
        c = lax.axis_index("c")
        s = lax.axis_index("s")
        r0 = s * STRIPE
        gsems = (g0, g1, g2, g3)
        ssems = (s0, s1, s2, s3)
        rows = (rows0_v, rows1_v, rows2_v, rows3_v)

        def preload(nchw, ch0):
            cp_s = pltpu.async_copy(edges_hbm.at[0, pl.ds(ch0, nchw)],
                                    src_v.at[pl.ds(0, nchw)], isem)
            cp_d = pltpu.async_copy(edges_hbm.at[1, pl.ds(ch0, nchw)],
                                    dst_v.at[pl.ds(0, nchw)], isem)
            cp_s.wait()
            cp_d.wait()

        with jax.named_scope("sc_preload"):
            @pl.when(c == 0)
            def _():
                preload(nch_c0, s * nch_c0)

            @pl.when(c == 1)
            def _():
                preload(nch_c1, NS * nch_c0 + s * nch_c1)

        def zero_body(r, _):
            for k in range(width // 16):
                rows0_v[r, pl.ds(k * 16, 16)] = jnp.zeros((16,), jnp.float32)
            return 0

        with jax.named_scope("sc_zerofill"):
            lax.fori_loop(0, CHUNK, zero_body, 0)
        for (o, n) in pieces:
            pltpu.async_copy(rows0_v.at[pl.ds(0, n)],
                             acc_sh.at[pl.ds(r0 + o, n)], zsem)

        @pl.when(s == NS - 1)
        def _():
            pltpu.async_copy(rows0_v.at[pl.ds(0, TAIL)],
                             acc_sh.at[pl.ds(NS * STRIPE, TAIL)], zsem)

        for (o, n) in pieces:
            pltpu.make_async_copy(rows0_v.at[pl.ds(0, n)],
                                  acc_sh.at[pl.ds(r0 + o, n)], zsem).wait()

        @pl.when(s == NS - 1)
        def _():
            pltpu.make_async_copy(rows0_v.at[pl.ds(0, TAIL)],
                                  acc_sh.at[pl.ds(NS * STRIPE, TAIL)],
                                  zsem).wait()

        with jax.named_scope("sc_barrier1"):
            plsc.subcore_barrier()

        def g_fire(j, b):
            pltpu.async_copy(tab_hbm.at[src_v.at[j]], rows[b], gsems[b])

        def g_wait(j, b):
            pltpu.make_async_copy(tab_hbm.at[src_v.at[j]], rows[b],
                                  gsems[b]).wait()

        def s_fire(j, b):
            pltpu.async_copy(rows[b], acc_sh.at[dst_v.at[j]], ssems[b],
                             add=True)

        def s_wait(j, b):
            pltpu.make_async_copy(rows[b], acc_sh.at[dst_v.at[j]],
                                  ssems[b]).wait()

        def ring(nchw):
            g_fire(0, 0)
            g_fire(1, 1)
            g_fire(2, 2)
            for j in range(4):
                g_wait(j, j % 4)
                s_fire(j, j % 4)
                if j >= 1:
                    s_wait(j - 1, (j - 1) % 4)
                g_fire(j + 3, (j + 3) % 4)

            def quad_body(g, _):
                for b in range(4):
                    j = g * 4 + b
                    g_wait(j, b)
                    s_fire(j, b)
                    s_wait(j - 1, (b - 1) % 4)
                    g_fire(j + 3, (b + 3) % 4)
                return 0

            lax.fori_loop(1, (nchw - 3) // 4, quad_body, 0)
            for j in range(nchw - 3, nchw):
                g_wait(j, j % 4)
                s_fire(j, j % 4)
                s_wait(j - 1, (j - 1) % 4)
            s_wait(nchw - 1, (nchw - 1) % 4)

        with jax.named_scope("sc_ring"):
            @pl.when(c == 0)
            def _():
                ring(nch_c0)

            @pl.when(c == 1)
            def _():
                ring(nch_c1)

        with jax.named_scope("sc_barrier2"):
            plsc.subcore_barrier()

        def in_hop(i):
            o, n = pieces[i]
            return (acc_sh.at[pl.ds(r0 + o, n)], rows[i % 2].at[pl.ds(0, n)])

        def out_hop(i):
            o, n = pieces[i]
            return (rows[i % 2].at[pl.ds(0, n)],
                    out_hbm.at[c, pl.ds(r0 + o, n)])

        scope_out = jax.named_scope("sc_copyout"); scope_out.__enter__()
        pltpu.async_copy(*in_hop(0), zsem)
        for i in range(len(pieces)):
            pltpu.make_async_copy(*in_hop(i), zsem).wait()
            pltpu.async_copy(*out_hop(i), ssems[i % 2])
            if i >= 1:
                pltpu.make_async_copy(*out_hop(i - 1), ssems[(i - 1) % 2]).wait()
            if i + 1 < len(pieces):
                pltpu.async_copy(*in_hop(i + 1), zsem)
        last = len(pieces) - 1
        pltpu.make_async_copy(*out_hop(last), ssems[last % 2]).wait()

        @pl.when(s == NS - 1)
        def _():
            pltpu.sync_copy(acc_sh.at[pl.ds(NS * STRIPE, TAIL)],
                            rows2_v.at[pl.ds(0, TAIL)])
            pltpu.sync_copy(rows2_v.at[pl.ds(0, TAIL)],
                            out_hbm.at[c, pl.ds(NS * STRIPE, TAIL)])
        scope_out.__exit__(None, None, None)

    return scat


_scatter1 = _make_scatter(W1P, 79, 79)
_scatter2 = _make_scatter(W2P, 79, 79)


def kernel(x, edge_index, W1_l, b1, W1_r, W2_l, b2, W2_r):
    padn = NCH_PAD * CHUNK - N_EDGES
    ar = jnp.arange(padn, dtype=jnp.int32)
    pad = jnp.concatenate(
        [(ar % N_NODES).reshape(1, padn),
         (N_NODES + ar).reshape(1, padn)], axis=0)
    edges3 = jnp.concatenate([edge_index, pad], axis=1).reshape(
        2, NCH_PAD, CHUNK)
    y1p, xr1 = _lin1(x, W1_l, W1_r, b1.reshape(1, HID_DIM))
    agg1p = _scatter1(edges3, y1p)
    y2p, r2, dinv = _lin2(agg1p, xr1, W2_l, W2_r, b2.reshape(1, OUT_DIM))
    agg2p = _scatter2(edges3, y2p)
    return _final(agg2p, r2, dinv)


# speedup vs baseline: 1.8163x; 1.1094x over previous
import functools

import jax
import jax.numpy as jnp
from jax import lax
from jax.experimental import pallas as pl
from jax.experimental.pallas import tpu as pltpu
from jax.experimental.pallas import tpu_sc as plsc

N_NODES = 10000
N_EDGES = 320000
IN_DIM = 128
HID_DIM = 64
OUT_DIM = 3

W1P = 80
W2P = 16
TABW = 128

NC = 2
NS = 16
NW = NC * NS
CHUNK = 128
NCH = N_EDGES // CHUNK
NCHW = -(-NCH // NW)
NCH_PAD = NW * NCHW
N_PAD_EDGES = NW * (-(-(N_EDGES // 128) // NW)) * 128 - N_EDGES
N_TAB = N_NODES + 8
STRIPE = 624
TAIL = N_NODES - NS * STRIPE


def _lin1_body(x_ref, wl_ref, wr_ref, b_ref, y1p_ref, xr1_ref):
    x = x_ref[...]
    y = lax.dot_general(x, wl_ref[...], (((1,), (1,)), ((), ())),
                        preferred_element_type=jnp.float32)
    n = x.shape[0]
    top = jnp.concatenate(
        [y, jnp.ones((n, 1), jnp.float32),
         jnp.zeros((n, W1P - HID_DIM - 1), jnp.float32)], axis=1)
    y1p_ref[...] = jnp.concatenate(
        [top, jnp.zeros((N_TAB - n, W1P), jnp.float32)], axis=0)
    xr1_ref[...] = lax.dot_general(x, wr_ref[...], (((1,), (1,)), ((), ())),
                                   preferred_element_type=jnp.float32) + b_ref[...]


def _lin1(x, wl, wr, b):
    return pl.pallas_call(
        _lin1_body,
        out_shape=(
            jax.ShapeDtypeStruct((N_TAB, W1P), jnp.float32),
            jax.ShapeDtypeStruct((N_NODES, HID_DIM), jnp.float32),
        ),
    )(x, wl, wr, b)


def _lin2_body(aggp_ref, xr1_ref, wl_ref, wr_ref, b_ref,
               y2p_ref, r2_ref, dinv_ref):
    agg = aggp_ref[0] + aggp_ref[1]
    deg = agg[:, HID_DIM:HID_DIM + 1]
    dinv = 1.0 / jnp.maximum(deg, 1.0)
    h = jnp.maximum(agg[:, :HID_DIM] * dinv + xr1_ref[...], 0.0)
    y2 = lax.dot_general(h, wl_ref[...], (((1,), (1,)), ((), ())),
                         preferred_element_type=jnp.float32)
    n = h.shape[0]
    top = jnp.concatenate(
        [y2, jnp.zeros((n, W2P - OUT_DIM), jnp.float32)], axis=1)
    y2p_ref[...] = jnp.concatenate(
        [top, jnp.zeros((N_TAB - n, W2P), jnp.float32)], axis=0)
    r2_ref[...] = lax.dot_general(h, wr_ref[...], (((1,), (1,)), ((), ())),
                                  preferred_element_type=jnp.float32) + b_ref[...]
    dinv_ref[...] = dinv


def _lin2(aggp, xr1, wl, wr, b):
    return pl.pallas_call(
        _lin2_body,
        out_shape=(
            jax.ShapeDtypeStruct((N_TAB, W2P), jnp.float32),
            jax.ShapeDtypeStruct((N_NODES, OUT_DIM), jnp.float32),
            jax.ShapeDtypeStruct((N_NODES, 1), jnp.float32),
        ),
    )(aggp, xr1, wl, wr, b)


def _final_body(aggp_ref, r2_ref, dinv_ref, o_ref):
    agg = aggp_ref[0] + aggp_ref[1]
    z = agg[:, :OUT_DIM] * dinv_ref[...] + r2_ref[...]
    m = jnp.max(z, axis=1, keepdims=True)
    lse = jnp.log(jnp.sum(jnp.exp(z - m), axis=1, keepdims=True)) + m
    o_ref[...] = z - lse


def _final(aggp, r2, dinv):
    return pl.pallas_call(
        _final_body,
        out_shape=jax.ShapeDtypeStruct((N_NODES, OUT_DIM), jnp.float32),
    )(aggp, r2, dinv)


def _make_scatter(width, nch_c0, nch_c1):
    assert nch_c0 % 4 == 3 and nch_c1 % 4 == 3
    assert NS * (nch_c0 + nch_c1) == NCH_PAD
    nmax = max(nch_c0, nch_c1)
    mesh = plsc.VectorSubcoreMesh(core_axis_name="c", subcore_axis_name="s")

    pieces = [(k * CHUNK, CHUNK) for k in range(STRIPE // CHUNK)]
    if STRIPE % CHUNK:
        pieces.append((STRIPE - STRIPE % CHUNK, STRIPE % CHUNK))

    @functools.partial(
        pl.kernel,
        mesh=mesh,
        compiler_params=pltpu.CompilerParams(use_tc_tiling_on_sc=False),
        out_type=jax.ShapeDtypeStruct((NC, N_NODES, TABW), jnp.float32),
        scratch_types=[
            pltpu.VMEM((nmax, CHUNK), jnp.int32),
            pltpu.VMEM((nmax, CHUNK), jnp.int32),
            pltpu.VMEM((CHUNK, width), jnp.float32),
            pltpu.VMEM((CHUNK, width), jnp.float32),
            pltpu.VMEM((CHUNK, width), jnp.float32),
            pltpu.VMEM((CHUNK, width), jnp.float32),
            pltpu.VMEM_SHARED((N_NODES + N_PAD_EDGES, width), jnp.float32),
            pltpu.SemaphoreType.DMA,
            pltpu.SemaphoreType.DMA,
            pltpu.SemaphoreType.DMA,
            pltpu.SemaphoreType.DMA,
            pltpu.SemaphoreType.DMA,
            pltpu.SemaphoreType.DMA,
            pltpu.SemaphoreType.DMA,
            pltpu.SemaphoreType.DMA,
            pltpu.SemaphoreType.DMA,
            pltpu.SemaphoreType.DMA,
        ],
    )
    def scat(edges_hbm, tab_hbm, out_hbm,
             src_v, dst_v, rows0_v, rows1_v, rows2_v, rows3_v, acc_sh,
             isem, zsem, g0, g1, g2, g3, s0, s1, s2, s3):
        c = lax.axis_index("c")
        s = lax.axis_index("s")
        r0 = s * STRIPE
        gsems = (g0, g1, g2, g3)
        ssems = (s0, s1, s2, s3)
        rows = (rows0_v, rows1_v, rows2_v, rows3_v)

        def preload(nchw, ch0):
            cp_s = pltpu.async_copy(edges_hbm.at[0, pl.ds(ch0, nchw)],
                                    src_v.at[pl.ds(0, nchw)], isem)
            cp_d = pltpu.async_copy(edges_hbm.at[1, pl.ds(ch0, nchw)],
                                    dst_v.at[pl.ds(0, nchw)], isem)
            cp_s.wait()
            cp_d.wait()

        with jax.named_scope("sc_preload"):
            @pl.when(c == 0)
            def _():
                preload(nch_c0, s * nch_c0)

            @pl.when(c == 1)
            def _():
                preload(nch_c1, NS * nch_c0 + s * nch_c1)

        def zero_body(r, _):
            for k in range(width // 16):
                rows0_v[r, pl.ds(k * 16, 16)] = jnp.zeros((16,), jnp.float32)
            return 0

        with jax.named_scope("sc_zerofill"):
            lax.fori_loop(0, CHUNK, zero_body, 0)
        for (o, n) in pieces:
            pltpu.async_copy(rows0_v.at[pl.ds(0, n)],
                             acc_sh.at[pl.ds(r0 + o, n)], zsem)

        @pl.when(s == NS - 1)
        def _():
            pltpu.async_copy(rows0_v.at[pl.ds(0, TAIL)],
                             acc_sh.at[pl.ds(NS * STRIPE, TAIL)], zsem)

        for (o, n) in pieces:
            pltpu.make_async_copy(rows0_v.at[pl.ds(0, n)],
                                  acc_sh.at[pl.ds(r0 + o, n)], zsem).wait()

        @pl.when(s == NS - 1)
        def _():
            pltpu.make_async_copy(rows0_v.at[pl.ds(0, TAIL)],
                                  acc_sh.at[pl.ds(NS * STRIPE, TAIL)],
                                  zsem).wait()

        with jax.named_scope("sc_barrier1"):
            plsc.subcore_barrier()

        def g_fire(j, b):
            pltpu.async_copy(tab_hbm.at[src_v.at[j]], rows[b], gsems[b])

        def g_wait(j, b):
            pltpu.make_async_copy(tab_hbm.at[src_v.at[j]], rows[b],
                                  gsems[b]).wait()

        def s_fire(j, b):
            pltpu.async_copy(rows[b], acc_sh.at[dst_v.at[j]], ssems[b],
                             add=True)

        def s_wait(j, b):
            pltpu.make_async_copy(rows[b], acc_sh.at[dst_v.at[j]],
                                  ssems[b]).wait()

        def ring(nchw):
            g_fire(0, 0)
            g_fire(1, 1)
            g_fire(2, 2)
            for j in range(4):
                g_wait(j, j % 4)
                s_fire(j, j % 4)
                if j >= 1:
                    s_wait(j - 1, (j - 1) % 4)
                g_fire(j + 3, (j + 3) % 4)

            def quad_body(g, _):
                for b in range(4):
                    j = g * 4 + b
                    g_wait(j, b)
                    s_fire(j, b)
                    s_wait(j - 1, (b - 1) % 4)
                    g_fire(j + 3, (b + 3) % 4)
                return 0

            lax.fori_loop(1, (nchw - 3) // 4, quad_body, 0)
            for j in range(nchw - 3, nchw):
                g_wait(j, j % 4)
                s_fire(j, j % 4)
                s_wait(j - 1, (j - 1) % 4)
            s_wait(nchw - 1, (nchw - 1) % 4)

        with jax.named_scope("sc_ring"):
            @pl.when(c == 0)
            def _():
                ring(nch_c0)

            @pl.when(c == 1)
            def _():
                ring(nch_c1)

        with jax.named_scope("sc_barrier2"):
            plsc.subcore_barrier()

        def in_hop(i):
            o, n = pieces[i]
            return (acc_sh.at[pl.ds(r0 + o, n)], rows[i % 2].at[pl.ds(0, n)])

        def out_hop(i):
            o, n = pieces[i]
            return (rows[i % 2].at[pl.ds(0, n)],
                    out_hbm.at[c, pl.ds(r0 + o, n), pl.ds(0, width)])

        scope_out = jax.named_scope("sc_copyout"); scope_out.__enter__()
        pltpu.async_copy(*in_hop(0), zsem)
        for i in range(len(pieces)):
            pltpu.make_async_copy(*in_hop(i), zsem).wait()
            pltpu.async_copy(*out_hop(i), ssems[i % 2])
            if i >= 1:
                pltpu.make_async_copy(*out_hop(i - 1), ssems[(i - 1) % 2]).wait()
            if i + 1 < len(pieces):
                pltpu.async_copy(*in_hop(i + 1), zsem)
        last = len(pieces) - 1
        pltpu.make_async_copy(*out_hop(last), ssems[last % 2]).wait()

        @pl.when(s == NS - 1)
        def _():
            pltpu.sync_copy(acc_sh.at[pl.ds(NS * STRIPE, TAIL)],
                            rows2_v.at[pl.ds(0, TAIL)])
            pltpu.sync_copy(rows2_v.at[pl.ds(0, TAIL)],
                            out_hbm.at[c, pl.ds(NS * STRIPE, TAIL),
                                       pl.ds(0, width)])
        scope_out.__exit__(None, None, None)

    return scat


_scatter1 = _make_scatter(W1P, 79, 79)
_scatter2 = _make_scatter(W2P, 79, 79)


def kernel(x, edge_index, W1_l, b1, W1_r, W2_l, b2, W2_r):
    padn = NCH_PAD * CHUNK - N_EDGES
    ar = jnp.arange(padn, dtype=jnp.int32)
    pad = jnp.concatenate(
        [(ar % N_NODES).reshape(1, padn),
         (N_NODES + ar).reshape(1, padn)], axis=0)
    edges3 = jnp.concatenate([edge_index, pad], axis=1).reshape(
        2, NCH_PAD, CHUNK)
    y1p, xr1 = _lin1(x, W1_l, W1_r, b1.reshape(1, HID_DIM))
    agg1p = _scatter1(edges3, y1p)
    y2p, r2, dinv = _lin2(agg1p, xr1, W2_l, W2_r, b2.reshape(1, OUT_DIM))
    agg2p = _scatter2(edges3, y2p)
    return _final(agg2p, r2, dinv)


# speedup vs baseline: 1.8560x; 1.0219x over previous
import functools

import numpy as np

import jax
import jax.numpy as jnp
from jax import lax
from jax.experimental import pallas as pl
from jax.experimental.pallas import tpu as pltpu
from jax.experimental.pallas import tpu_sc as plsc

N_NODES = 10000
N_EDGES = 320000
IN_DIM = 128
HID_DIM = 64
OUT_DIM = 3

W1P = 80
W2P = 16
TABW = 128

NC = 2
NS = 16
NW = NC * NS
CHUNK = 128
NCH = N_EDGES // CHUNK
NCHW = -(-NCH // NW)
NCH_PAD = NW * NCHW
N_PAD_EDGES = NW * (-(-(N_EDGES // 128) // NW)) * 128 - N_EDGES
N_TAB = N_NODES + 8
STRIPE = 624
TAIL = N_NODES - NS * STRIPE


def _lin1_body(x_ref, wl_ref, y1p_ref):
    x = x_ref[...]
    y = lax.dot_general(x, wl_ref[...], (((1,), (1,)), ((), ())),
                        preferred_element_type=jnp.float32)
    n = x.shape[0]
    top = jnp.concatenate(
        [y, jnp.ones((n, 1), jnp.float32),
         jnp.zeros((n, W1P - HID_DIM - 1), jnp.float32)], axis=1)
    y1p_ref[...] = jnp.concatenate(
        [top, jnp.zeros((N_TAB - n, W1P), jnp.float32)], axis=0)


def _lin1(x, wl):
    return pl.pallas_call(
        _lin1_body,
        out_shape=jax.ShapeDtypeStruct((N_TAB, W1P), jnp.float32),
    )(x, wl)


def _xr1_body(x_ref, wr_ref, b_ref, xr1_ref):
    xr1_ref[...] = lax.dot_general(
        x_ref[...], wr_ref[...], (((1,), (1,)), ((), ())),
        preferred_element_type=jnp.float32) + b_ref[...]


def _xr1(x, wr, b):
    return pl.pallas_call(
        _xr1_body,
        out_shape=jax.ShapeDtypeStruct((N_NODES, HID_DIM), jnp.float32),
    )(x, wr, b)


def _lin2_body(aggp_ref, xr1_ref, wl_ref, y2p_ref, h_ref, dinv_ref):
    agg = aggp_ref[0] + aggp_ref[1]
    deg = agg[:, HID_DIM:HID_DIM + 1]
    dinv = 1.0 / jnp.maximum(deg, 1.0)
    h = jnp.maximum(agg[:, :HID_DIM] * dinv + xr1_ref[...], 0.0)
    y2 = lax.dot_general(h, wl_ref[...], (((1,), (1,)), ((), ())),
                         preferred_element_type=jnp.float32)
    n = h.shape[0]
    top = jnp.concatenate(
        [y2, jnp.zeros((n, W2P - OUT_DIM), jnp.float32)], axis=1)
    y2p_ref[...] = jnp.concatenate(
        [top, jnp.zeros((N_TAB - n, W2P), jnp.float32)], axis=0)
    h_ref[...] = h
    dinv_ref[...] = dinv


def _lin2(aggp, xr1, wl):
    return pl.pallas_call(
        _lin2_body,
        out_shape=(
            jax.ShapeDtypeStruct((N_TAB, W2P), jnp.float32),
            jax.ShapeDtypeStruct((N_NODES, HID_DIM), jnp.float32),
            jax.ShapeDtypeStruct((N_NODES, 1), jnp.float32),
        ),
    )(aggp, xr1, wl)


def _r2(h, wr, b):
    return pl.pallas_call(
        _xr1_body,
        out_shape=jax.ShapeDtypeStruct((N_NODES, OUT_DIM), jnp.float32),
    )(h, wr, b)


def _final_body(aggp_ref, r2_ref, dinv_ref, o_ref):
    agg = aggp_ref[0] + aggp_ref[1]
    z = agg[:, :OUT_DIM] * dinv_ref[...] + r2_ref[...]
    m = jnp.max(z, axis=1, keepdims=True)
    lse = jnp.log(jnp.sum(jnp.exp(z - m), axis=1, keepdims=True)) + m
    o_ref[...] = z - lse


def _final(aggp, r2, dinv):
    return pl.pallas_call(
        _final_body,
        out_shape=jax.ShapeDtypeStruct((N_NODES, OUT_DIM), jnp.float32),
    )(aggp, r2, dinv)


def _make_scatter(width, nch_c0, nch_c1):
    assert nch_c0 % 4 == 3 and nch_c1 % 4 == 3
    assert NS * (nch_c0 + nch_c1) == NCH_PAD
    nmax = max(nch_c0, nch_c1)
    mesh = plsc.VectorSubcoreMesh(core_axis_name="c", subcore_axis_name="s")

    pieces = [(k * CHUNK, CHUNK) for k in range(STRIPE // CHUNK)]
    if STRIPE % CHUNK:
        pieces.append((STRIPE - STRIPE % CHUNK, STRIPE % CHUNK))

    @functools.partial(
        pl.kernel,
        mesh=mesh,
        compiler_params=pltpu.CompilerParams(use_tc_tiling_on_sc=False),
        out_type=jax.ShapeDtypeStruct((NC, N_NODES, TABW), jnp.float32),
        scratch_types=[
            pltpu.VMEM((nmax, CHUNK), jnp.int32),
            pltpu.VMEM((nmax, CHUNK), jnp.int32),
            pltpu.VMEM((CHUNK, width), jnp.float32),
            pltpu.VMEM((CHUNK, width), jnp.float32),
            pltpu.VMEM((CHUNK, width), jnp.float32),
            pltpu.VMEM((CHUNK, width), jnp.float32),
            pltpu.VMEM_SHARED((N_NODES + N_PAD_EDGES, width), jnp.float32),
            pltpu.SemaphoreType.DMA,
            pltpu.SemaphoreType.DMA,
            pltpu.SemaphoreType.DMA,
            pltpu.SemaphoreType.DMA,
            pltpu.SemaphoreType.DMA,
            pltpu.SemaphoreType.DMA,
            pltpu.SemaphoreType.DMA,
            pltpu.SemaphoreType.DMA,
            pltpu.SemaphoreType.DMA,
            pltpu.SemaphoreType.DMA,
        ],
    )
    def scat(edges_hbm, tab_hbm, out_hbm,
             src_v, dst_v, rows0_v, rows1_v, rows2_v, rows3_v, acc_sh,
             isem, zsem, g0, g1, g2, g3, s0, s1, s2, s3):
        c = lax.axis_index("c")
        s = lax.axis_index("s")
        r0 = s * STRIPE
        gsems = (g0, g1, g2, g3)
        ssems = (s0, s1, s2, s3)
        rows = (rows0_v, rows1_v, rows2_v, rows3_v)

        def preload(nchw, ch0):
            cp_s = pltpu.async_copy(edges_hbm.at[0, pl.ds(ch0, nchw)],
                                    src_v.at[pl.ds(0, nchw)], isem)
            cp_d = pltpu.async_copy(edges_hbm.at[1, pl.ds(ch0, nchw)],
                                    dst_v.at[pl.ds(0, nchw)], isem)
            cp_s.wait()
            cp_d.wait()

        @pl.when(c == 0)
        def _():
            preload(nch_c0, s * nch_c0)

        @pl.when(c == 1)
        def _():
            preload(nch_c1, NS * nch_c0 + s * nch_c1)

        def zero_body(r, _):
            for k in range(width // 16):
                rows0_v[r, pl.ds(k * 16, 16)] = jnp.zeros((16,), jnp.float32)
            return 0

        lax.fori_loop(0, CHUNK, zero_body, 0)
        for (o, n) in pieces:
            pltpu.async_copy(rows0_v.at[pl.ds(0, n)],
                             acc_sh.at[pl.ds(r0 + o, n)], zsem)

        @pl.when(s == NS - 1)
        def _():
            pltpu.async_copy(rows0_v.at[pl.ds(0, TAIL)],
                             acc_sh.at[pl.ds(NS * STRIPE, TAIL)], zsem)

        for (o, n) in pieces:
            pltpu.make_async_copy(rows0_v.at[pl.ds(0, n)],
                                  acc_sh.at[pl.ds(r0 + o, n)], zsem).wait()

        @pl.when(s == NS - 1)
        def _():
            pltpu.make_async_copy(rows0_v.at[pl.ds(0, TAIL)],
                                  acc_sh.at[pl.ds(NS * STRIPE, TAIL)],
                                  zsem).wait()

        plsc.subcore_barrier()

        def g_fire(j, b):
            pltpu.async_copy(tab_hbm.at[src_v.at[j]], rows[b], gsems[b])

        def g_wait(j, b):
            pltpu.make_async_copy(tab_hbm.at[src_v.at[j]], rows[b],
                                  gsems[b]).wait()

        def s_fire(j, b):
            pltpu.async_copy(rows[b], acc_sh.at[dst_v.at[j]], ssems[b],
                             add=True)

        def s_wait(j, b):
            pltpu.make_async_copy(rows[b], acc_sh.at[dst_v.at[j]],
                                  ssems[b]).wait()

        def ring(nchw):
            g_fire(0, 0)
            g_fire(1, 1)
            g_fire(2, 2)
            for j in range(4):
                g_wait(j, j % 4)
                s_fire(j, j % 4)
                if j >= 1:
                    s_wait(j - 1, (j - 1) % 4)
                g_fire(j + 3, (j + 3) % 4)

            def quad_body(g, _):
                for b in range(4):
                    j = g * 4 + b
                    g_wait(j, b)
                    s_fire(j, b)
                    s_wait(j - 1, (b - 1) % 4)
                    g_fire(j + 3, (b + 3) % 4)
                return 0

            lax.fori_loop(1, (nchw - 3) // 4, quad_body, 0)
            for j in range(nchw - 3, nchw):
                g_wait(j, j % 4)
                s_fire(j, j % 4)
                s_wait(j - 1, (j - 1) % 4)
            s_wait(nchw - 1, (nchw - 1) % 4)

        @pl.when(c == 0)
        def _():
            ring(nch_c0)

        @pl.when(c == 1)
        def _():
            ring(nch_c1)

        plsc.subcore_barrier()

        def in_hop(i):
            o, n = pieces[i]
            return (acc_sh.at[pl.ds(r0 + o, n)], rows[i % 2].at[pl.ds(0, n)])

        def out_hop(i):
            o, n = pieces[i]
            return (rows[i % 2].at[pl.ds(0, n)],
                    out_hbm.at[c, pl.ds(r0 + o, n), pl.ds(0, width)])

        pltpu.async_copy(*in_hop(0), zsem)
        for i in range(len(pieces)):
            pltpu.make_async_copy(*in_hop(i), zsem).wait()
            pltpu.async_copy(*out_hop(i), ssems[i % 2])
            if i >= 1:
                pltpu.make_async_copy(*out_hop(i - 1), ssems[(i - 1) % 2]).wait()
            if i + 1 < len(pieces):
                pltpu.async_copy(*in_hop(i + 1), zsem)
        last = len(pieces) - 1
        pltpu.make_async_copy(*out_hop(last), ssems[last % 2]).wait()

        @pl.when(s == NS - 1)
        def _():
            pltpu.sync_copy(acc_sh.at[pl.ds(NS * STRIPE, TAIL)],
                            rows2_v.at[pl.ds(0, TAIL)])
            pltpu.sync_copy(rows2_v.at[pl.ds(0, TAIL)],
                            out_hbm.at[c, pl.ds(NS * STRIPE, TAIL),
                                       pl.ds(0, width)])

    return scat


_scatter1 = _make_scatter(W1P, 79, 79)
_scatter2 = _make_scatter(W2P, 79, 79)


def kernel(x, edge_index, W1_l, b1, W1_r, W2_l, b2, W2_r):
    padn = NCH_PAD * CHUNK - N_EDGES
    ar = np.arange(padn, dtype=np.int32)
    pad = jnp.asarray(np.stack([ar % N_NODES,
                                N_NODES + ar]))
    edges3 = jnp.concatenate([edge_index, pad], axis=1).reshape(
        2, NCH_PAD, CHUNK)
    y1p = _lin1(x, W1_l)
    agg1p = _scatter1(edges3, y1p)
    xr1 = _xr1(x, W1_r, b1.reshape(1, HID_DIM))
    y2p, h, dinv = _lin2(agg1p, xr1, W2_l)
    agg2p = _scatter2(edges3, y2p)
    r2 = _r2(h, W2_r, b2.reshape(1, OUT_DIM))
    return _final(agg2p, r2, dinv)
